# baseline (speedup 1.0000x reference)
import functools

import jax
import jax.numpy as jnp
from jax import lax
from jax.experimental import pallas as pl
from jax.experimental.pallas import tpu as pltpu
from jax.experimental.pallas import tpu_sc as plsc

B = 16
S = 512
EMB = 512
HID = 1024

T_CHUNK = 64
SPLITS = (128, 384)

SC_NC = 2
SC_NS = 16
SC_NW = SC_NC * SC_NS
GCHUNK = 64


def _sc_gather(table, idx):
    n, d = idx.shape[0], table.shape[1]
    rows_per_w = n // SC_NW
    n_gchunks = max(rows_per_w // GCHUNK, 1)
    gchunk = rows_per_w // n_gchunks
    mesh = plsc.VectorSubcoreMesh(core_axis_name="c", subcore_axis_name="s")

    @functools.partial(
        pl.kernel,
        mesh=mesh,
        out_type=jax.ShapeDtypeStruct((n, d), table.dtype),
        scratch_types=[
            pltpu.VMEM((rows_per_w,), jnp.int32),
            pltpu.VMEM((gchunk, d), table.dtype),
            pltpu.VMEM((gchunk, d), table.dtype),
            pltpu.SemaphoreType.DMA,
            pltpu.SemaphoreType.DMA,
            pltpu.SemaphoreType.DMA,
            pltpu.SemaphoreType.DMA,
        ],
    )
    def gather_kernel(table_hbm, idx_hbm, out_hbm, idx_v, rows0, rows1,
                      g0, g1, w0, w1):
        wid = lax.axis_index("s") * SC_NC + lax.axis_index("c")
        base = wid * rows_per_w
        pltpu.sync_copy(idx_hbm.at[pl.ds(base, rows_per_w)], idx_v)

        rows = (rows0, rows1)
        gsem = (g0, g1)
        wsem = (w0, w1)

        def start_gather(j):
            return pltpu.async_copy(
                table_hbm.at[idx_v.at[pl.ds(j * gchunk, gchunk)]],
                rows[j % 2], gsem[j % 2])

        def start_write(j):
            return pltpu.async_copy(
                rows[j % 2], out_hbm.at[pl.ds(base + j * gchunk, gchunk)],
                wsem[j % 2])

        if n_gchunks == 1:
            pltpu.async_copy(table_hbm.at[idx_v], rows0, g0).wait()
            pltpu.async_copy(rows0, out_hbm.at[pl.ds(base, gchunk)], w0).wait()
        else:
            gathers = [start_gather(0), start_gather(1)]
            writes = [None, None]
            for j in range(n_gchunks):
                gathers[j % 2].wait()
                writes[j % 2] = start_write(j)
                nxt = j + 2
                if nxt < n_gchunks:
                    writes[nxt % 2].wait()
                    gathers[nxt % 2] = start_gather(nxt)
            writes[(n_gchunks - 2) % 2].wait()
            writes[(n_gchunks - 1) % 2].wait()

    return gather_kernel(table, idx)


def _matmul_kernel(x_ref, w_ref, o_ref):
    o_ref[...] = jnp.dot(x_ref[...], w_ref[...],
                         preferred_element_type=jnp.float32)


def _matmul(x, w, block_m=1024):
    m, k = x.shape
    n = w.shape[1]
    return pl.pallas_call(
        _matmul_kernel,
        grid=(m // block_m,),
        in_specs=[
            pl.BlockSpec((block_m, k), lambda g: (g, 0)),
            pl.BlockSpec((k, n), lambda g: (0, 0)),
        ],
        out_specs=pl.BlockSpec((block_m, n), lambda g: (g, 0)),
        out_shape=jax.ShapeDtypeStruct((m, n), jnp.float32),
    )(x, w)


def _recurrence_kernel(a_ref, m_ref, w_ref, b_ref, hin_ref, hall_ref,
                       hfin_ref, h_scratch):
    g = pl.program_id(0)

    @pl.when(g == 0)
    def _():
        h_scratch[...] = hin_ref[...]

    w = w_ref[...]
    b = b_ref[...]

    def step(t, h):
        a = a_ref[pl.ds(t * B, B), :]
        h_new = jnp.tanh(a + jnp.dot(h, w, preferred_element_type=jnp.float32)
                         + b)
        hall_ref[pl.ds(t * B, B), :] = h_new
        m = m_ref[pl.ds(t * B, B), :]
        return jnp.where(m > 0, h_new, h)

    h = lax.fori_loop(0, T_CHUNK, step, h_scratch[...])
    h_scratch[...] = h

    @pl.when(g == pl.num_programs(0) - 1)
    def _():
        hfin_ref[...] = h


def _recurrence(a, mask, w_hh, b, h_in):
    rows = a.shape[0]
    return pl.pallas_call(
        _recurrence_kernel,
        grid=(rows // (T_CHUNK * B),),
        in_specs=[
            pl.BlockSpec((T_CHUNK * B, HID), lambda g: (g, 0)),
            pl.BlockSpec((T_CHUNK * B, 1), lambda g: (g, 0)),
            pl.BlockSpec((HID, HID), lambda g: (0, 0)),
            pl.BlockSpec((1, HID), lambda g: (0, 0)),
            pl.BlockSpec((B, HID), lambda g: (0, 0)),
        ],
        out_specs=[
            pl.BlockSpec((T_CHUNK * B, HID), lambda g: (g, 0)),
            pl.BlockSpec((B, HID), lambda g: (0, 0)),
        ],
        out_shape=[
            jax.ShapeDtypeStruct((rows, HID), jnp.float32),
            jax.ShapeDtypeStruct((B, HID), jnp.float32),
        ],
        scratch_shapes=[pltpu.VMEM((B, HID), jnp.float32)],
    )(a, mask, w_hh, b.reshape(1, HID), h_in)


def kernel(x, embedding, W_ih0, W_hh0, b0, W_ih1, W_hh1, b1):
    xt = x.T
    idx = xt.reshape(-1).astype(jnp.int32)
    mask = (xt != 0).astype(jnp.float32).reshape(S * B, 1)

    offs = [0]
    for s_c in SPLITS:
        offs.append(offs[-1] + s_c * B)
    xe = [_sc_gather(embedding, lax.slice(idx, (offs[c],), (offs[c + 1],)))
          for c in range(len(SPLITS))]

    h0 = jnp.zeros((B, HID), jnp.float32)
    h1 = jnp.zeros((B, HID), jnp.float32)
    for c in range(len(SPLITS)):
        mask_c = lax.slice(mask, (offs[c], 0), (offs[c + 1], 1))
        a0_c = _matmul(xe[c], W_ih0)
        h0all_c, h0 = _recurrence(a0_c, mask_c, W_hh0, b0, h0)
        a1_c = _matmul(h0all_c, W_ih1)
        _, h1 = _recurrence(a1_c, mask_c, W_hh1, b1, h1)
    return h1


# speedup vs baseline: 1.0368x; 1.0368x over previous
import functools

import jax
import jax.numpy as jnp
from jax import lax
from jax.experimental import pallas as pl
from jax.experimental.pallas import tpu as pltpu
from jax.experimental.pallas import tpu_sc as plsc

B = 16
S = 512
EMB = 512
HID = 1024

T_CHUNK = 128
SPLITS = (256, 256)

SC_NC = 2
SC_NS = 16
SC_NW = SC_NC * SC_NS
GCHUNK = 64


def _sc_gather(table, idx):
    n, d = idx.shape[0], table.shape[1]
    rows_per_w = n // SC_NW
    n_gchunks = max(rows_per_w // GCHUNK, 1)
    gchunk = rows_per_w // n_gchunks
    mesh = plsc.VectorSubcoreMesh(core_axis_name="c", subcore_axis_name="s")

    @functools.partial(
        pl.kernel,
        mesh=mesh,
        out_type=jax.ShapeDtypeStruct((n, d), table.dtype),
        scratch_types=[
            pltpu.VMEM((rows_per_w,), jnp.int32),
            pltpu.VMEM((gchunk, d), table.dtype),
            pltpu.VMEM((gchunk, d), table.dtype),
            pltpu.SemaphoreType.DMA,
            pltpu.SemaphoreType.DMA,
            pltpu.SemaphoreType.DMA,
            pltpu.SemaphoreType.DMA,
        ],
    )
    def gather_kernel(table_hbm, idx_hbm, out_hbm, idx_v, rows0, rows1,
                      g0, g1, w0, w1):
        wid = lax.axis_index("s") * SC_NC + lax.axis_index("c")
        base = wid * rows_per_w
        pltpu.sync_copy(idx_hbm.at[pl.ds(base, rows_per_w)], idx_v)

        rows = (rows0, rows1)
        gsem = (g0, g1)
        wsem = (w0, w1)

        def start_gather(j):
            return pltpu.async_copy(
                table_hbm.at[idx_v.at[pl.ds(j * gchunk, gchunk)]],
                rows[j % 2], gsem[j % 2])

        def start_write(j):
            return pltpu.async_copy(
                rows[j % 2], out_hbm.at[pl.ds(base + j * gchunk, gchunk)],
                wsem[j % 2])

        if n_gchunks == 1:
            pltpu.async_copy(table_hbm.at[idx_v], rows0, g0).wait()
            pltpu.async_copy(rows0, out_hbm.at[pl.ds(base, gchunk)], w0).wait()
        else:
            gathers = [start_gather(0), start_gather(1)]
            writes = [None, None]
            for j in range(n_gchunks):
                gathers[j % 2].wait()
                writes[j % 2] = start_write(j)
                nxt = j + 2
                if nxt < n_gchunks:
                    writes[nxt % 2].wait()
                    gathers[nxt % 2] = start_gather(nxt)
            writes[(n_gchunks - 2) % 2].wait()
            writes[(n_gchunks - 1) % 2].wait()

    return gather_kernel(table, idx)


def _matmul_kernel(x_ref, w_ref, o_ref):
    o_ref[...] = jnp.dot(x_ref[...], w_ref[...],
                         preferred_element_type=jnp.float32)


def _matmul(x, w, block_m=1024):
    m, k = x.shape
    n = w.shape[1]
    return pl.pallas_call(
        _matmul_kernel,
        grid=(m // block_m,),
        in_specs=[
            pl.BlockSpec((block_m, k), lambda g: (g, 0)),
            pl.BlockSpec((k, n), lambda g: (0, 0)),
        ],
        out_specs=pl.BlockSpec((block_m, n), lambda g: (g, 0)),
        out_shape=jax.ShapeDtypeStruct((m, n), jnp.float32),
    )(x, w)


def _recurrence_kernel(a_ref, m_ref, w_ref, b_ref, hin_ref, hall_ref,
                       hfin_ref, h_scratch):
    g = pl.program_id(0)

    @pl.when(g == 0)
    def _():
        h_scratch[...] = hin_ref[...]

    w = w_ref[...]
    b = b_ref[...]

    def step(t, h):
        a = a_ref[pl.ds(t * B, B), :]
        h_new = jnp.tanh(a + jnp.dot(h, w, preferred_element_type=jnp.float32)
                         + b)
        hall_ref[pl.ds(t * B, B), :] = h_new
        m = m_ref[pl.ds(t * B, B), :]
        return jnp.where(m > 0, h_new, h)

    h = lax.fori_loop(0, T_CHUNK, step, h_scratch[...])
    h_scratch[...] = h

    @pl.when(g == pl.num_programs(0) - 1)
    def _():
        hfin_ref[...] = h


def _recurrence(a, mask, w_hh, b, h_in):
    rows = a.shape[0]
    return pl.pallas_call(
        _recurrence_kernel,
        grid=(rows // (T_CHUNK * B),),
        in_specs=[
            pl.BlockSpec((T_CHUNK * B, HID), lambda g: (g, 0)),
            pl.BlockSpec((T_CHUNK * B, 1), lambda g: (g, 0)),
            pl.BlockSpec((HID, HID), lambda g: (0, 0)),
            pl.BlockSpec((1, HID), lambda g: (0, 0)),
            pl.BlockSpec((B, HID), lambda g: (0, 0)),
        ],
        out_specs=[
            pl.BlockSpec((T_CHUNK * B, HID), lambda g: (g, 0)),
            pl.BlockSpec((B, HID), lambda g: (0, 0)),
        ],
        out_shape=[
            jax.ShapeDtypeStruct((rows, HID), jnp.float32),
            jax.ShapeDtypeStruct((B, HID), jnp.float32),
        ],
        scratch_shapes=[pltpu.VMEM((B, HID), jnp.float32)],
    )(a, mask, w_hh, b.reshape(1, HID), h_in)


def kernel(x, embedding, W_ih0, W_hh0, b0, W_ih1, W_hh1, b1):
    xt = x.T
    idx = xt.reshape(-1).astype(jnp.int32)
    mask = (xt != 0).astype(jnp.float32).reshape(S * B, 1)

    offs = [0]
    for s_c in SPLITS:
        offs.append(offs[-1] + s_c * B)
    xe = [_sc_gather(embedding, lax.slice(idx, (offs[c],), (offs[c + 1],)))
          for c in range(len(SPLITS))]

    h0 = jnp.zeros((B, HID), jnp.float32)
    h1 = jnp.zeros((B, HID), jnp.float32)
    for c in range(len(SPLITS)):
        mask_c = lax.slice(mask, (offs[c], 0), (offs[c + 1], 1))
        a0_c = _matmul(xe[c], W_ih0)
        h0all_c, h0 = _recurrence(a0_c, mask_c, W_hh0, b0, h0)
        a1_c = _matmul(h0all_c, W_ih1)
        _, h1 = _recurrence(a1_c, mask_c, W_hh1, b1, h1)
    return h1


# speedup vs baseline: 1.0393x; 1.0024x over previous
import functools

import jax
import jax.numpy as jnp
from jax import lax
from jax.experimental import pallas as pl
from jax.experimental.pallas import tpu as pltpu
from jax.experimental.pallas import tpu_sc as plsc

B = 16
S = 512
EMB = 512
HID = 1024

T_CHUNK = 64
SPLITS = (192, 320)

SC_NC = 2
SC_NS = 16
SC_NW = SC_NC * SC_NS
GCHUNK = 64


def _sc_gather(table, idx):
    n, d = idx.shape[0], table.shape[1]
    rows_per_w = n // SC_NW
    n_gchunks = max(rows_per_w // GCHUNK, 1)
    gchunk = rows_per_w // n_gchunks
    mesh = plsc.VectorSubcoreMesh(core_axis_name="c", subcore_axis_name="s")

    @functools.partial(
        pl.kernel,
        mesh=mesh,
        out_type=jax.ShapeDtypeStruct((n, d), table.dtype),
        scratch_types=[
            pltpu.VMEM((rows_per_w,), jnp.int32),
            pltpu.VMEM((gchunk, d), table.dtype),
            pltpu.VMEM((gchunk, d), table.dtype),
            pltpu.SemaphoreType.DMA,
            pltpu.SemaphoreType.DMA,
            pltpu.SemaphoreType.DMA,
            pltpu.SemaphoreType.DMA,
        ],
    )
    def gather_kernel(table_hbm, idx_hbm, out_hbm, idx_v, rows0, rows1,
                      g0, g1, w0, w1):
        wid = lax.axis_index("s") * SC_NC + lax.axis_index("c")
        base = wid * rows_per_w
        pltpu.sync_copy(idx_hbm.at[pl.ds(base, rows_per_w)], idx_v)

        rows = (rows0, rows1)
        gsem = (g0, g1)
        wsem = (w0, w1)

        def start_gather(j):
            return pltpu.async_copy(
                table_hbm.at[idx_v.at[pl.ds(j * gchunk, gchunk)]],
                rows[j % 2], gsem[j % 2])

        def start_write(j):
            return pltpu.async_copy(
                rows[j % 2], out_hbm.at[pl.ds(base + j * gchunk, gchunk)],
                wsem[j % 2])

        if n_gchunks == 1:
            pltpu.async_copy(table_hbm.at[idx_v], rows0, g0).wait()
            pltpu.async_copy(rows0, out_hbm.at[pl.ds(base, gchunk)], w0).wait()
        else:
            gathers = [start_gather(0), start_gather(1)]
            writes = [None, None]
            for j in range(n_gchunks):
                gathers[j % 2].wait()
                writes[j % 2] = start_write(j)
                nxt = j + 2
                if nxt < n_gchunks:
                    writes[nxt % 2].wait()
                    gathers[nxt % 2] = start_gather(nxt)
            writes[(n_gchunks - 2) % 2].wait()
            writes[(n_gchunks - 1) % 2].wait()

    return gather_kernel(table, idx)


def _matmul_kernel(x_ref, w_ref, o_ref):
    o_ref[...] = jnp.dot(x_ref[...], w_ref[...],
                         preferred_element_type=jnp.float32)


def _matmul(x, w, block_m=1024):
    m, k = x.shape
    n = w.shape[1]
    return pl.pallas_call(
        _matmul_kernel,
        grid=(m // block_m,),
        in_specs=[
            pl.BlockSpec((block_m, k), lambda g: (g, 0)),
            pl.BlockSpec((k, n), lambda g: (0, 0)),
        ],
        out_specs=pl.BlockSpec((block_m, n), lambda g: (g, 0)),
        out_shape=jax.ShapeDtypeStruct((m, n), jnp.float32),
    )(x, w)


def _recurrence_kernel(a_ref, m_ref, w_ref, b_ref, hin_ref, hall_ref,
                       hfin_ref, h_scratch):
    g = pl.program_id(0)

    @pl.when(g == 0)
    def _():
        h_scratch[...] = hin_ref[...]

    w = w_ref[...]
    b = b_ref[...]

    def step(t, h):
        a = a_ref[pl.ds(t * B, B), :]
        h_new = jnp.tanh(a + jnp.dot(h, w, preferred_element_type=jnp.float32)
                         + b)
        hall_ref[pl.ds(t * B, B), :] = h_new
        m = m_ref[pl.ds(t * B, B), :]
        return jnp.where(m > 0, h_new, h)

    h = lax.fori_loop(0, T_CHUNK, step, h_scratch[...])
    h_scratch[...] = h

    @pl.when(g == pl.num_programs(0) - 1)
    def _():
        hfin_ref[...] = h


def _recurrence(a, mask, w_hh, b, h_in):
    rows = a.shape[0]
    return pl.pallas_call(
        _recurrence_kernel,
        grid=(rows // (T_CHUNK * B),),
        in_specs=[
            pl.BlockSpec((T_CHUNK * B, HID), lambda g: (g, 0)),
            pl.BlockSpec((T_CHUNK * B, 1), lambda g: (g, 0)),
            pl.BlockSpec((HID, HID), lambda g: (0, 0)),
            pl.BlockSpec((1, HID), lambda g: (0, 0)),
            pl.BlockSpec((B, HID), lambda g: (0, 0)),
        ],
        out_specs=[
            pl.BlockSpec((T_CHUNK * B, HID), lambda g: (g, 0)),
            pl.BlockSpec((B, HID), lambda g: (0, 0)),
        ],
        out_shape=[
            jax.ShapeDtypeStruct((rows, HID), jnp.float32),
            jax.ShapeDtypeStruct((B, HID), jnp.float32),
        ],
        scratch_shapes=[pltpu.VMEM((B, HID), jnp.float32)],
    )(a, mask, w_hh, b.reshape(1, HID), h_in)


def kernel(x, embedding, W_ih0, W_hh0, b0, W_ih1, W_hh1, b1):
    xt = x.T
    idx = xt.reshape(-1).astype(jnp.int32)
    mask = (xt != 0).astype(jnp.float32).reshape(S * B, 1)

    offs = [0]
    for s_c in SPLITS:
        offs.append(offs[-1] + s_c * B)
    xe = [_sc_gather(embedding, lax.slice(idx, (offs[c],), (offs[c + 1],)))
          for c in range(len(SPLITS))]

    h0 = jnp.zeros((B, HID), jnp.float32)
    h1 = jnp.zeros((B, HID), jnp.float32)
    for c in range(len(SPLITS)):
        mask_c = lax.slice(mask, (offs[c], 0), (offs[c + 1], 1))
        a0_c = _matmul(xe[c], W_ih0)
        h0all_c, h0 = _recurrence(a0_c, mask_c, W_hh0, b0, h0)
        a1_c = _matmul(h0all_c, W_ih1)
        _, h1 = _recurrence(a1_c, mask_c, W_hh1, b1, h1)
    return h1


# speedup vs baseline: 1.0719x; 1.0314x over previous
import functools

import jax
import jax.numpy as jnp
from jax import lax
from jax.experimental import pallas as pl
from jax.experimental.pallas import tpu as pltpu
from jax.experimental.pallas import tpu_sc as plsc

B = 16
S = 512
EMB = 512
HID = 1024

T_CHUNK = 32
SPLITS = (256, 256)

SC_NC = 2
SC_NS = 16
SC_NW = SC_NC * SC_NS
GCHUNK = 64


def _sc_gather(table, idx):
    n, d = idx.shape[0], table.shape[1]
    rows_per_w = n // SC_NW
    n_gchunks = max(rows_per_w // GCHUNK, 1)
    gchunk = rows_per_w // n_gchunks
    mesh = plsc.VectorSubcoreMesh(core_axis_name="c", subcore_axis_name="s")

    @functools.partial(
        pl.kernel,
        mesh=mesh,
        out_type=jax.ShapeDtypeStruct((n, d), table.dtype),
        scratch_types=[
            pltpu.VMEM((rows_per_w,), jnp.int32),
            pltpu.VMEM((gchunk, d), table.dtype),
            pltpu.VMEM((gchunk, d), table.dtype),
            pltpu.SemaphoreType.DMA,
            pltpu.SemaphoreType.DMA,
            pltpu.SemaphoreType.DMA,
            pltpu.SemaphoreType.DMA,
        ],
    )
    def gather_kernel(table_hbm, idx_hbm, out_hbm, idx_v, rows0, rows1,
                      g0, g1, w0, w1):
        wid = lax.axis_index("s") * SC_NC + lax.axis_index("c")
        base = wid * rows_per_w
        pltpu.sync_copy(idx_hbm.at[pl.ds(base, rows_per_w)], idx_v)

        rows = (rows0, rows1)
        gsem = (g0, g1)
        wsem = (w0, w1)

        def start_gather(j):
            return pltpu.async_copy(
                table_hbm.at[idx_v.at[pl.ds(j * gchunk, gchunk)]],
                rows[j % 2], gsem[j % 2])

        def start_write(j):
            return pltpu.async_copy(
                rows[j % 2], out_hbm.at[pl.ds(base + j * gchunk, gchunk)],
                wsem[j % 2])

        if n_gchunks == 1:
            pltpu.async_copy(table_hbm.at[idx_v], rows0, g0).wait()
            pltpu.async_copy(rows0, out_hbm.at[pl.ds(base, gchunk)], w0).wait()
        else:
            gathers = [start_gather(0), start_gather(1)]
            writes = [None, None]
            for j in range(n_gchunks):
                gathers[j % 2].wait()
                writes[j % 2] = start_write(j)
                nxt = j + 2
                if nxt < n_gchunks:
                    writes[nxt % 2].wait()
                    gathers[nxt % 2] = start_gather(nxt)
            writes[(n_gchunks - 2) % 2].wait()
            writes[(n_gchunks - 1) % 2].wait()

    return gather_kernel(table, idx)


def _matmul_kernel(x_ref, w_ref, o_ref):
    o_ref[...] = jnp.dot(x_ref[...], w_ref[...],
                         preferred_element_type=jnp.float32)


def _matmul(x, w, block_m=1024):
    m, k = x.shape
    n = w.shape[1]
    return pl.pallas_call(
        _matmul_kernel,
        grid=(m // block_m,),
        in_specs=[
            pl.BlockSpec((block_m, k), lambda g: (g, 0)),
            pl.BlockSpec((k, n), lambda g: (0, 0)),
        ],
        out_specs=pl.BlockSpec((block_m, n), lambda g: (g, 0)),
        out_shape=jax.ShapeDtypeStruct((m, n), jnp.float32),
    )(x, w)


def _recurrence_kernel(a_ref, m_ref, w_ref, b_ref, hin_ref, hall_ref,
                       hfin_ref, h_scratch):
    g = pl.program_id(0)

    @pl.when(g == 0)
    def _():
        h_scratch[...] = hin_ref[...]

    w = w_ref[...]
    b = b_ref[...]

    def step(t, h):
        a = a_ref[pl.ds(t * B, B), :]
        h_new = jnp.tanh(a + jnp.dot(h, w, preferred_element_type=jnp.float32)
                         + b)
        hall_ref[pl.ds(t * B, B), :] = h_new
        m = m_ref[pl.ds(t * B, B), :]
        return jnp.where(m > 0, h_new, h)

    h = lax.fori_loop(0, T_CHUNK, step, h_scratch[...])
    h_scratch[...] = h

    @pl.when(g == pl.num_programs(0) - 1)
    def _():
        hfin_ref[...] = h


def _recurrence(a, mask, w_hh, b, h_in):
    rows = a.shape[0]
    return pl.pallas_call(
        _recurrence_kernel,
        grid=(rows // (T_CHUNK * B),),
        in_specs=[
            pl.BlockSpec((T_CHUNK * B, HID), lambda g: (g, 0)),
            pl.BlockSpec((T_CHUNK * B, 1), lambda g: (g, 0)),
            pl.BlockSpec((HID, HID), lambda g: (0, 0)),
            pl.BlockSpec((1, HID), lambda g: (0, 0)),
            pl.BlockSpec((B, HID), lambda g: (0, 0)),
        ],
        out_specs=[
            pl.BlockSpec((T_CHUNK * B, HID), lambda g: (g, 0)),
            pl.BlockSpec((B, HID), lambda g: (0, 0)),
        ],
        out_shape=[
            jax.ShapeDtypeStruct((rows, HID), jnp.float32),
            jax.ShapeDtypeStruct((B, HID), jnp.float32),
        ],
        scratch_shapes=[pltpu.VMEM((B, HID), jnp.float32)],
    )(a, mask, w_hh, b.reshape(1, HID), h_in)


def kernel(x, embedding, W_ih0, W_hh0, b0, W_ih1, W_hh1, b1):
    xt = x.T
    idx = xt.reshape(-1).astype(jnp.int32)
    mask = (xt != 0).astype(jnp.float32).reshape(S * B, 1)

    offs = [0]
    for s_c in SPLITS:
        offs.append(offs[-1] + s_c * B)
    xe = [_sc_gather(embedding, lax.slice(idx, (offs[c],), (offs[c + 1],)))
          for c in range(len(SPLITS))]

    h0 = jnp.zeros((B, HID), jnp.float32)
    h1 = jnp.zeros((B, HID), jnp.float32)
    for c in range(len(SPLITS)):
        mask_c = lax.slice(mask, (offs[c], 0), (offs[c + 1], 1))
        a0_c = _matmul(xe[c], W_ih0)
        h0all_c, h0 = _recurrence(a0_c, mask_c, W_hh0, b0, h0)
        a1_c = _matmul(h0all_c, W_ih1)
        _, h1 = _recurrence(a1_c, mask_c, W_hh1, b1, h1)
    return h1
